# baseline (speedup 1.0000x reference)
import functools

import jax
import jax.numpy as jnp
from jax.experimental import pallas as pl
from jax.experimental.pallas import tpu as pltpu

_H = 56
_W = 56
_NP = _H * _W
_TAPS = tuple((dy, dx) for dy in (-1, 0, 1) for dx in (-1, 0, 1))


def _gelu(v):
    return 0.5 * v * (1.0 + jax.lax.erf(v * 0.7071067811865476))


def _conv_taps(z, m_ref):
    acc = jnp.zeros((2, _NP), jnp.float32)
    for t, (dy, dx) in enumerate(_TAPS):
        off = dy * _W + dx
        zt = z[2 * t:2 * t + 2]
        if off:
            zt = jnp.roll(zt, -off, axis=1)
        if dy or dx:
            zt = zt * m_ref[t:t + 1, :]
        acc = acc + zt
    return acc


def _frame_kernel(xlo_ref, xhi_ref, w1_ref, b1_ref, w2_ref, b2_ref, m_ref,
                  o_ref, *, c_half, frames):
    for f in range(frames):
        xlo = xlo_ref[0, f, 0]
        xhi = xhi_ref[0, f, 0]

        z = (jnp.dot(w1_ref[:, :c_half], xlo, preferred_element_type=jnp.float32)
             + jnp.dot(w1_ref[:, c_half:], xhi, preferred_element_type=jnp.float32))
        g = _gelu(_conv_taps(z, m_ref) + b1_ref[...])

        z2 = jnp.dot(w2_ref[...], g, preferred_element_type=jnp.float32)
        g2 = _gelu(_conv_taps(z2, m_ref) + b2_ref[...])

        p0 = jax.nn.sigmoid(g2[0:1] - g2[1:2])
        o_ref[0, :, f, 0] = p0 * xlo + (1.0 - p0) * xhi


@jax.jit
def kernel(x, type, w1, b1, w2, b2):
    del type
    b, t, c, h, w = x.shape
    c_half = c // 2
    xr = x.reshape(b, t, 2, c_half, _NP)

    w1s = w1.transpose(2, 3, 0, 1).reshape(9 * 2, c)
    w2s = w2.transpose(2, 3, 0, 1).reshape(9 * 2, 2)
    b1c = b1.reshape(2, 1)
    b2c = b2.reshape(2, 1)

    ys = jnp.arange(_H)
    xs = jnp.arange(_W)
    rows = []
    for dy, dx in _TAPS:
        vy = ((ys + dy) >= 0) & ((ys + dy) < _H)
        vx = ((xs + dx) >= 0) & ((xs + dx) < _W)
        rows.append((vy[:, None] & vx[None, :]).reshape(_NP))
    masks = jnp.stack(rows).astype(jnp.float32)

    frames = 4
    out = pl.pallas_call(
        functools.partial(_frame_kernel, c_half=c_half, frames=frames),
        grid=(b, t // frames),
        in_specs=[
            pl.BlockSpec((1, frames, 1, c_half, _NP), lambda i, j: (i, j, 0, 0, 0)),
            pl.BlockSpec((1, frames, 1, c_half, _NP), lambda i, j: (i, j, 1, 0, 0)),
            pl.BlockSpec((9 * 2, c), lambda i, j: (0, 0)),
            pl.BlockSpec((2, 1), lambda i, j: (0, 0)),
            pl.BlockSpec((9 * 2, 2), lambda i, j: (0, 0)),
            pl.BlockSpec((2, 1), lambda i, j: (0, 0)),
            pl.BlockSpec((9, _NP), lambda i, j: (0, 0)),
        ],
        out_specs=pl.BlockSpec((1, c_half, frames, 1, _NP),
                               lambda i, j: (i, 0, j, 0, 0)),
        out_shape=jax.ShapeDtypeStruct((b, c_half, t, 1, _NP), x.dtype),
        compiler_params=pltpu.CompilerParams(vmem_limit_bytes=100 * 1024 * 1024),
    )(xr, xr, w1s, b1c, w2s, b2c, masks)
    return out.reshape(b, c_half, t, h, w)


# speedup vs baseline: 1.6867x; 1.6867x over previous
import functools

import jax
import jax.numpy as jnp
from jax.experimental import pallas as pl
from jax.experimental.pallas import tpu as pltpu

_H = 56
_W = 56
_NP = _H * _W
_TAPS = tuple((dy, dx) for dy in (-1, 0, 1) for dx in (-1, 0, 1))


def _gelu(v):
    return 0.5 * v * (1.0 + jax.lax.erf(v * 0.7071067811865476))


def _conv_taps(z, m_ref):
    acc = jnp.zeros((2, _NP), jnp.float32)
    for t, (dy, dx) in enumerate(_TAPS):
        off = dy * _W + dx
        zt = z[2 * t:2 * t + 2]
        if off:
            zt = jnp.roll(zt, -off, axis=1)
        if dy or dx:
            zt = zt * m_ref[t:t + 1, :]
        acc = acc + zt
    return acc


def _frame_kernel(x_ref, w1_ref, b1_ref, w2_ref, b2_ref, m_ref,
                  o_ref, *, c_half, frames):
    for f in range(frames):
        x = x_ref[0, f]

        z = jnp.dot(w1_ref[...], x, preferred_element_type=jnp.float32)
        g = _gelu(_conv_taps(z, m_ref) + b1_ref[...])

        z2 = jnp.dot(w2_ref[...], g, preferred_element_type=jnp.float32)
        g2 = _gelu(_conv_taps(z2, m_ref) + b2_ref[...])

        p0 = jax.nn.sigmoid(g2[0:1] - g2[1:2])
        o_ref[0, :, f, 0] = p0 * x[:c_half] + (1.0 - p0) * x[c_half:]


@jax.jit
def kernel(x, type, w1, b1, w2, b2):
    del type
    b, t, c, h, w = x.shape
    c_half = c // 2
    xr = x.reshape(b, t, c, _NP)

    w1s = w1.transpose(2, 3, 0, 1).reshape(9 * 2, c)
    w2s = w2.transpose(2, 3, 0, 1).reshape(9 * 2, 2)
    b1c = b1.reshape(2, 1)
    b2c = b2.reshape(2, 1)

    ys = jnp.arange(_H)
    xs = jnp.arange(_W)
    rows = []
    for dy, dx in _TAPS:
        vy = ((ys + dy) >= 0) & ((ys + dy) < _H)
        vx = ((xs + dx) >= 0) & ((xs + dx) < _W)
        rows.append((vy[:, None] & vx[None, :]).reshape(_NP))
    masks = jnp.stack(rows).astype(jnp.float32)

    frames = 4
    out = pl.pallas_call(
        functools.partial(_frame_kernel, c_half=c_half, frames=frames),
        grid=(b, t // frames),
        in_specs=[
            pl.BlockSpec((1, frames, c, _NP), lambda i, j: (i, j, 0, 0)),
            pl.BlockSpec((9 * 2, c), lambda i, j: (0, 0)),
            pl.BlockSpec((2, 1), lambda i, j: (0, 0)),
            pl.BlockSpec((9 * 2, 2), lambda i, j: (0, 0)),
            pl.BlockSpec((2, 1), lambda i, j: (0, 0)),
            pl.BlockSpec((9, _NP), lambda i, j: (0, 0)),
        ],
        out_specs=pl.BlockSpec((1, c_half, frames, 1, _NP),
                               lambda i, j: (i, 0, j, 0, 0)),
        out_shape=jax.ShapeDtypeStruct((b, c_half, t, 1, _NP), x.dtype),
        compiler_params=pltpu.CompilerParams(vmem_limit_bytes=100 * 1024 * 1024),
    )(xr, w1s, b1c, w2s, b2c, masks)
    return out.reshape(b, c_half, t, h, w)
